# baseline (speedup 1.0000x reference)
import functools

import jax
import jax.numpy as jnp
from jax import lax
from jax.experimental import pallas as pl
from jax.experimental.pallas import tpu as pltpu
from jax.experimental.pallas import tpu_sc as plsc

N = 10000
D = 128
E = 160000
NC = 2
NS = 16
NP = 10240
RPT = NP // NS
EPT = E // NS
K = 125
C = EPT // K
NBUF = 2
NI = 4
CW = 16
BM = 2000


@functools.cache
def _sc_aggregate(counts):
  mesh = plsc.VectorSubcoreMesh(core_axis_name="c", subcore_axis_name="s")

  out_type = [jax.ShapeDtypeStruct((NC, NP, D), jnp.float32)]
  scratch = [
      pltpu.VMEM_SHARED((NP, D), jnp.float32),
      [pltpu.VMEM((2, K), jnp.int32) for _ in range(NI)],
      [pltpu.VMEM((K, D), jnp.float32) for _ in range(NBUF)],
      [pltpu.SemaphoreType.DMA for _ in range(NI)],
      [pltpu.SemaphoreType.DMA for _ in range(NBUF)],
      [pltpu.SemaphoreType.DMA for _ in range(NBUF)],
  ]
  if counts:
    out_type.append(jax.ShapeDtypeStruct((NC, NP, CW), jnp.float32))
    scratch += [
        pltpu.VMEM_SHARED((NP, CW), jnp.float32),
        pltpu.VMEM((K, CW), jnp.float32),
        pltpu.SemaphoreType.DMA,
    ]

  @functools.partial(
      pl.kernel,
      out_type=tuple(out_type),
      mesh=mesh,
      scratch_types=scratch,
      compiler_params=pltpu.CompilerParams(use_tc_tiling_on_sc=False),
  )
  def agg(x, idx, zeros, *rest):
    if counts:
      (zeros_c, ones, out, out_c, acc, idxr, rows, isem, gsem, ssem,
       cacc, ones_v, csem) = rest
    else:
      out, acc, idxr, rows, isem, gsem, ssem = rest
    c = lax.axis_index("c")
    s = lax.axis_index("s")
    r0 = s * RPT
    pltpu.sync_copy(zeros, acc.at[pl.ds(r0, RPT)])
    if counts:
      pltpu.sync_copy(zeros_c, cacc.at[pl.ds(r0, RPT)])
      pltpu.sync_copy(ones, ones_v)
    for q in range(NI):
      pltpu.async_copy(idx.at[c, s, q], idxr[q], isem[q])
    for b in range(NBUF):
      pltpu.make_async_copy(idx.at[c, s, b], idxr[b], isem[b]).wait()
      pltpu.async_copy(x.at[idxr[b].at[0]], rows[b], gsem[b])
    plsc.subcore_barrier()

    def body(i, carry):
      for u in range(NI):
        j = i * NI + u
        b = u % NBUF
        pltpu.make_async_copy(x.at[idxr[u].at[0]], rows[b], gsem[b]).wait()
        if counts:
          cd = pltpu.async_copy(ones_v, cacc.at[idxr[u].at[1]], csem,
                                add=True)
        sd = pltpu.async_copy(rows[b], acc.at[idxr[u].at[1]], ssem[b],
                              add=True)
        sd.wait()
        if counts:
          cd.wait()

        @pl.when(j + NI < C)
        def _():
          pltpu.async_copy(idx.at[c, s, j + NI], idxr[u], isem[u])

        q2 = (u + NBUF) % NI

        @pl.when(j + NBUF < C)
        def _():
          pltpu.make_async_copy(idx.at[c, s, j + NBUF], idxr[q2],
                                isem[q2]).wait()
          pltpu.async_copy(x.at[idxr[q2].at[0]], rows[b], gsem[b])

      return carry

    lax.fori_loop(0, C // NI, body, 0)
    plsc.subcore_barrier()
    pltpu.sync_copy(acc.at[pl.ds(r0, RPT)], out.at[c, pl.ds(r0, RPT)])
    if counts:
      pltpu.sync_copy(cacc.at[pl.ds(r0, RPT)], out_c.at[c, pl.ds(r0, RPT)])

  return agg


def _dot(a, b):
  return jnp.dot(a, b, preferred_element_type=jnp.float32,
                 precision=lax.Precision.DEFAULT)


def _root_body(x_ref, wroot_ref, b_ref, out_ref):
  out_ref[...] = _dot(x_ref[...], wroot_ref[...]) + b_ref[...]


def _tc_root(x, wroot, b):
  return pl.pallas_call(
      _root_body,
      grid=(N // BM,),
      in_specs=[
          pl.BlockSpec((BM, D), lambda i: (i, 0)),
          pl.BlockSpec((D, D), lambda i: (0, 0)),
          pl.BlockSpec((1, D), lambda i: (0, 0)),
      ],
      out_specs=pl.BlockSpec((BM, D), lambda i: (i, 0)),
      out_shape=jax.ShapeDtypeStruct((N, D), jnp.float32),
  )(x, wroot, b)


def _combine_body(relu):
  def body(root_ref, agg_ref, cnt_ref, wr_ref, out_ref):
    m0 = agg_ref[0] / jnp.maximum(cnt_ref[0][:, 0:1], 1.0)
    m1 = agg_ref[1] / jnp.maximum(cnt_ref[1][:, 0:1], 1.0)
    acc = root_ref[...] + _dot(m0, wr_ref[0]) + _dot(m1, wr_ref[1])
    out_ref[...] = jnp.maximum(acc, 0.0) if relu else acc

  return body


def _tc_combine(relu, root, agg, cnt, wr):
  return pl.pallas_call(
      _combine_body(relu),
      grid=(N // BM,),
      in_specs=[
          pl.BlockSpec((BM, D), lambda i: (i, 0)),
          pl.BlockSpec((NC, BM, D), lambda i: (0, i, 0)),
          pl.BlockSpec((NC, BM, CW), lambda i: (0, i, 0)),
          pl.BlockSpec((NC, D, D), lambda i: (0, 0, 0)),
      ],
      out_specs=pl.BlockSpec((BM, D), lambda i: (i, 0)),
      out_shape=jax.ShapeDtypeStruct((N, D), jnp.float32),
  )(root, agg, cnt, wr)


def kernel(x, edge_index_r0, edge_index_r1, Wr0, Wroot0, b0, Wr1, Wroot1, b1):
  x = jnp.asarray(x, jnp.float32)
  idx = jnp.stack([edge_index_r0, edge_index_r1]).astype(jnp.int32)
  idx = idx.reshape(NC, 2, NS, C, K).transpose(0, 2, 3, 1, 4)

  zeros = jnp.zeros((RPT, D), jnp.float32)
  agg1, cnt = _sc_aggregate(True)(x, idx, zeros,
                                  jnp.zeros((RPT, CW), jnp.float32),
                                  jnp.ones((K, CW), jnp.float32))
  root0 = _tc_root(x, Wroot0, b0.reshape(1, D))
  h1 = _tc_combine(True, root0, agg1, cnt, Wr0)
  (agg2,) = _sc_aggregate(False)(h1, idx, zeros)
  root1 = _tc_root(h1, Wroot1, b1.reshape(1, D))
  return _tc_combine(False, root1, agg2, cnt, Wr1)


# speedup vs baseline: 1.0305x; 1.0305x over previous
import functools

import jax
import jax.numpy as jnp
from jax import lax
from jax.experimental import pallas as pl
from jax.experimental.pallas import tpu as pltpu
from jax.experimental.pallas import tpu_sc as plsc

N = 10000
D = 128
E = 160000
NC = 2
NS = 16
NP = 10240
RPT = NP // NS
EPT = E // NS
K = 125
C = EPT // K
NBUF = 2
NI = 4
CW = 16
BM = 2000


@functools.cache
def _sc_aggregate(counts):
  mesh = plsc.VectorSubcoreMesh(core_axis_name="c", subcore_axis_name="s")

  out_type = [jax.ShapeDtypeStruct((NC, NP, D), jnp.float32)]
  scratch = [
      pltpu.VMEM_SHARED((NP, D), jnp.float32),
      [pltpu.VMEM((2, K), jnp.int32) for _ in range(NI)],
      [pltpu.VMEM((K, D), jnp.float32) for _ in range(NBUF)],
      [pltpu.SemaphoreType.DMA for _ in range(NI)],
      [pltpu.SemaphoreType.DMA for _ in range(NBUF)],
      [pltpu.SemaphoreType.DMA for _ in range(NBUF)],
  ]
  if counts:
    out_type.append(jax.ShapeDtypeStruct((NC, NP, CW), jnp.float32))
    scratch += [
        pltpu.VMEM_SHARED((NP, CW), jnp.float32),
        pltpu.VMEM((K, CW), jnp.float32),
        pltpu.SemaphoreType.DMA,
    ]

  @functools.partial(
      pl.kernel,
      out_type=tuple(out_type),
      mesh=mesh,
      scratch_types=scratch,
      compiler_params=pltpu.CompilerParams(use_tc_tiling_on_sc=False),
  )
  def agg(x, idx, zeros, *rest):
    if counts:
      (zeros_c, ones, out, out_c, acc, idxr, rows, isem, gsem, ssem,
       cacc, ones_v, csem) = rest
    else:
      out, acc, idxr, rows, isem, gsem, ssem = rest
    c = lax.axis_index("c")
    s = lax.axis_index("s")
    r0 = s * RPT

    def fire_idx(q, j):
      pltpu.async_copy(idx.at[c, 0, s, j], idxr[q].at[0], isem[q])
      pltpu.async_copy(idx.at[c, 1, s, j], idxr[q].at[1], isem[q])

    def wait_idx(q, j):
      pltpu.make_async_copy(idx.at[c, 0, s, j], idxr[q].at[0], isem[q]).wait()
      pltpu.make_async_copy(idx.at[c, 1, s, j], idxr[q].at[1], isem[q]).wait()

    pltpu.sync_copy(zeros, acc.at[pl.ds(r0, RPT)])
    if counts:
      pltpu.sync_copy(zeros_c, cacc.at[pl.ds(r0, RPT)])
      pltpu.sync_copy(ones, ones_v)
    for q in range(NI):
      fire_idx(q, q)
    for b in range(NBUF):
      wait_idx(b, b)
      pltpu.async_copy(x.at[idxr[b].at[0]], rows[b], gsem[b])
    plsc.subcore_barrier()

    def body(i, carry):
      for u in range(NI):
        j = i * NI + u
        b = u % NBUF
        pltpu.make_async_copy(x.at[idxr[u].at[0]], rows[b], gsem[b]).wait()
        if counts:
          cd = pltpu.async_copy(ones_v, cacc.at[idxr[u].at[1]], csem,
                                add=True)
        sd = pltpu.async_copy(rows[b], acc.at[idxr[u].at[1]], ssem[b],
                              add=True)
        sd.wait()
        if counts:
          cd.wait()

        @pl.when(j + NI < C)
        def _():
          fire_idx(u, j + NI)

        q2 = (u + NBUF) % NI

        @pl.when(j + NBUF < C)
        def _():
          wait_idx(q2, j + NBUF)
          pltpu.async_copy(x.at[idxr[q2].at[0]], rows[b], gsem[b])

      return carry

    lax.fori_loop(0, C // NI, body, 0)
    plsc.subcore_barrier()
    pltpu.sync_copy(acc.at[pl.ds(r0, RPT)], out.at[c, pl.ds(r0, RPT)])
    if counts:
      pltpu.sync_copy(cacc.at[pl.ds(r0, RPT)], out_c.at[c, pl.ds(r0, RPT)])

  return agg


def _dot(a, b):
  return jnp.dot(a, b, preferred_element_type=jnp.float32,
                 precision=lax.Precision.DEFAULT)


def _tc1_body(x_ref, agg_ref, cnt_ref, wr_ref, wroot_ref, b_ref, h_ref):
  m0 = agg_ref[0] / jnp.maximum(cnt_ref[0][:, 0:1], 1.0)
  m1 = agg_ref[1] / jnp.maximum(cnt_ref[1][:, 0:1], 1.0)
  acc = _dot(x_ref[...], wroot_ref[...]) + b_ref[...]
  acc = acc + _dot(m0, wr_ref[0]) + _dot(m1, wr_ref[1])
  h_ref[...] = jnp.maximum(acc, 0.0)


def _tc2_body(h_ref, agg_ref, cnt_ref, wr_ref, wroot_ref, b_ref, out_ref):
  m0 = agg_ref[0] / jnp.maximum(cnt_ref[0][:, 0:1], 1.0)
  m1 = agg_ref[1] / jnp.maximum(cnt_ref[1][:, 0:1], 1.0)
  acc = _dot(h_ref[...], wroot_ref[...]) + b_ref[...]
  out_ref[...] = acc + _dot(m0, wr_ref[0]) + _dot(m1, wr_ref[1])


def _tc_layer(body, relu_unused):
  return pl.pallas_call(
      body,
      grid=(N // BM,),
      in_specs=[
          pl.BlockSpec((BM, D), lambda i: (i, 0)),
          pl.BlockSpec((NC, BM, D), lambda i: (0, i, 0)),
          pl.BlockSpec((NC, BM, CW), lambda i: (0, i, 0)),
          pl.BlockSpec((NC, D, D), lambda i: (0, 0, 0)),
          pl.BlockSpec((D, D), lambda i: (0, 0)),
          pl.BlockSpec((1, D), lambda i: (0, 0)),
      ],
      out_specs=pl.BlockSpec((BM, D), lambda i: (i, 0)),
      out_shape=jax.ShapeDtypeStruct((N, D), jnp.float32),
  )


def kernel(x, edge_index_r0, edge_index_r1, Wr0, Wroot0, b0, Wr1, Wroot1, b1):
  x = jnp.asarray(x, jnp.float32)
  idx = jnp.stack([edge_index_r0, edge_index_r1]).astype(jnp.int32)
  idx = idx.reshape(NC, 2, NS, C, K)

  zeros = jnp.zeros((RPT, D), jnp.float32)
  agg1, cnt = _sc_aggregate(True)(x, idx, zeros,
                                  jnp.zeros((RPT, CW), jnp.float32),
                                  jnp.ones((K, CW), jnp.float32))
  h1 = _tc_layer(_tc1_body, True)(x, agg1, cnt, Wr0, Wroot0,
                                  b0.reshape(1, D))
  (agg2,) = _sc_aggregate(False)(h1, idx, zeros)
  return _tc_layer(_tc2_body, False)(h1, agg2, cnt, Wr1, Wroot1,
                                     b1.reshape(1, D))


# speedup vs baseline: 1.0460x; 1.0150x over previous
import functools

import jax
import jax.numpy as jnp
from jax import lax
from jax.experimental import pallas as pl
from jax.experimental.pallas import tpu as pltpu
from jax.experimental.pallas import tpu_sc as plsc

N = 10000
D = 128
E = 160000
NC = 2
NS = 16
NP = 10240
RPT = NP // NS
EPT = E // NS
K = 50
C = EPT // K
NBUF = 4
NI = 8
CW = 16
BM = 2000


@functools.cache
def _sc_aggregate(counts):
  mesh = plsc.VectorSubcoreMesh(core_axis_name="c", subcore_axis_name="s")

  out_type = [jax.ShapeDtypeStruct((NC, NP, D), jnp.float32)]
  scratch = [
      pltpu.VMEM_SHARED((NP, D), jnp.float32),
      [pltpu.VMEM((2, K), jnp.int32) for _ in range(NI)],
      [pltpu.VMEM((K, D), jnp.float32) for _ in range(NBUF)],
      [pltpu.SemaphoreType.DMA for _ in range(NI)],
      [pltpu.SemaphoreType.DMA for _ in range(NBUF)],
      [pltpu.SemaphoreType.DMA for _ in range(NBUF)],
  ]
  if counts:
    out_type.append(jax.ShapeDtypeStruct((NC, NP, CW), jnp.float32))
    scratch += [
        pltpu.VMEM_SHARED((NP, CW), jnp.float32),
        pltpu.VMEM((K, CW), jnp.float32),
        pltpu.SemaphoreType.DMA,
    ]

  @functools.partial(
      pl.kernel,
      out_type=tuple(out_type),
      mesh=mesh,
      scratch_types=scratch,
      compiler_params=pltpu.CompilerParams(use_tc_tiling_on_sc=False),
  )
  def agg(x, idx, zeros, *rest):
    if counts:
      (zeros_c, ones, out, out_c, acc, idxr, rows, isem, gsem, ssem,
       cacc, ones_v, csem) = rest
    else:
      out, acc, idxr, rows, isem, gsem, ssem = rest
    c = lax.axis_index("c")
    s = lax.axis_index("s")
    r0 = s * RPT

    def fire_idx(q, j):
      pltpu.async_copy(idx.at[c, 0, s, j], idxr[q].at[0], isem[q])
      pltpu.async_copy(idx.at[c, 1, s, j], idxr[q].at[1], isem[q])

    def wait_idx(q, j):
      pltpu.make_async_copy(idx.at[c, 0, s, j], idxr[q].at[0], isem[q]).wait()
      pltpu.make_async_copy(idx.at[c, 1, s, j], idxr[q].at[1], isem[q]).wait()

    pltpu.sync_copy(zeros, acc.at[pl.ds(r0, RPT)])
    if counts:
      pltpu.sync_copy(zeros_c, cacc.at[pl.ds(r0, RPT)])
      pltpu.sync_copy(ones, ones_v)
    for q in range(NI):
      fire_idx(q, q)
    for b in range(NBUF):
      wait_idx(b, b)
      pltpu.async_copy(x.at[idxr[b].at[0]], rows[b], gsem[b])
    plsc.subcore_barrier()

    def body(i, carry):
      for u in range(NI):
        j = i * NI + u
        b = u % NBUF
        pltpu.make_async_copy(x.at[idxr[u].at[0]], rows[b], gsem[b]).wait()
        if counts:
          cd = pltpu.async_copy(ones_v, cacc.at[idxr[u].at[1]], csem,
                                add=True)
        sd = pltpu.async_copy(rows[b], acc.at[idxr[u].at[1]], ssem[b],
                              add=True)
        sd.wait()
        if counts:
          cd.wait()

        @pl.when(j + NI < C)
        def _():
          fire_idx(u, j + NI)

        q2 = (u + NBUF) % NI

        @pl.when(j + NBUF < C)
        def _():
          wait_idx(q2, j + NBUF)
          pltpu.async_copy(x.at[idxr[q2].at[0]], rows[b], gsem[b])

      return carry

    lax.fori_loop(0, C // NI, body, 0)
    plsc.subcore_barrier()
    pltpu.sync_copy(acc.at[pl.ds(r0, RPT)], out.at[c, pl.ds(r0, RPT)])
    if counts:
      pltpu.sync_copy(cacc.at[pl.ds(r0, RPT)], out_c.at[c, pl.ds(r0, RPT)])

  return agg


def _dot(a, b):
  return jnp.dot(a, b, preferred_element_type=jnp.float32,
                 precision=lax.Precision.DEFAULT)


def _tc1_body(x_ref, agg_ref, cnt_ref, wr_ref, wroot_ref, b_ref, h_ref):
  m0 = agg_ref[0] / jnp.maximum(cnt_ref[0][:, 0:1], 1.0)
  m1 = agg_ref[1] / jnp.maximum(cnt_ref[1][:, 0:1], 1.0)
  acc = _dot(x_ref[...], wroot_ref[...]) + b_ref[...]
  acc = acc + _dot(m0, wr_ref[0]) + _dot(m1, wr_ref[1])
  h_ref[...] = jnp.maximum(acc, 0.0)


def _tc2_body(h_ref, agg_ref, cnt_ref, wr_ref, wroot_ref, b_ref, out_ref):
  m0 = agg_ref[0] / jnp.maximum(cnt_ref[0][:, 0:1], 1.0)
  m1 = agg_ref[1] / jnp.maximum(cnt_ref[1][:, 0:1], 1.0)
  acc = _dot(h_ref[...], wroot_ref[...]) + b_ref[...]
  out_ref[...] = acc + _dot(m0, wr_ref[0]) + _dot(m1, wr_ref[1])


def _tc_layer(body, relu_unused):
  return pl.pallas_call(
      body,
      grid=(N // BM,),
      in_specs=[
          pl.BlockSpec((BM, D), lambda i: (i, 0)),
          pl.BlockSpec((NC, BM, D), lambda i: (0, i, 0)),
          pl.BlockSpec((NC, BM, CW), lambda i: (0, i, 0)),
          pl.BlockSpec((NC, D, D), lambda i: (0, 0, 0)),
          pl.BlockSpec((D, D), lambda i: (0, 0)),
          pl.BlockSpec((1, D), lambda i: (0, 0)),
      ],
      out_specs=pl.BlockSpec((BM, D), lambda i: (i, 0)),
      out_shape=jax.ShapeDtypeStruct((N, D), jnp.float32),
  )


def kernel(x, edge_index_r0, edge_index_r1, Wr0, Wroot0, b0, Wr1, Wroot1, b1):
  x = jnp.asarray(x, jnp.float32)
  idx = jnp.stack([edge_index_r0, edge_index_r1]).astype(jnp.int32)
  idx = idx.reshape(NC, 2, NS, C, K)

  zeros = jnp.zeros((RPT, D), jnp.float32)
  agg1, cnt = _sc_aggregate(True)(x, idx, zeros,
                                  jnp.zeros((RPT, CW), jnp.float32),
                                  jnp.ones((K, CW), jnp.float32))
  h1 = _tc_layer(_tc1_body, True)(x, agg1, cnt, Wr0, Wroot0,
                                  b0.reshape(1, D))
  (agg2,) = _sc_aggregate(False)(h1, idx, zeros)
  return _tc_layer(_tc2_body, False)(h1, agg2, cnt, Wr1, Wroot1,
                                     b1.reshape(1, D))


# speedup vs baseline: 1.0513x; 1.0050x over previous
import functools

import jax
import jax.numpy as jnp
from jax import lax
from jax.experimental import pallas as pl
from jax.experimental.pallas import tpu as pltpu
from jax.experimental.pallas import tpu_sc as plsc

N = 10000
D = 128
E = 160000
NC = 2
NS = 16
NP = 10240
RPT = NP // NS
EPT = E // NS
K = 50
C = EPT // K
NBUF = 4
NI = 8
CW = 16
BM = 2000


@functools.cache
def _sc_aggregate(counts):
  mesh = plsc.VectorSubcoreMesh(core_axis_name="c", subcore_axis_name="s")

  out_type = [jax.ShapeDtypeStruct((NC, NP, D), jnp.float32)]
  scratch = [
      pltpu.VMEM_SHARED((NP, D), jnp.float32),
      [pltpu.VMEM((2, K), jnp.int32) for _ in range(NI)],
      [pltpu.VMEM((K, D), jnp.float32) for _ in range(NBUF)],
      [pltpu.SemaphoreType.DMA for _ in range(NI)],
      [pltpu.SemaphoreType.DMA for _ in range(NBUF)],
      [pltpu.SemaphoreType.DMA for _ in range(NBUF)],
  ]
  if counts:
    out_type.append(jax.ShapeDtypeStruct((NC, NP, CW), jnp.float32))
    scratch += [
        pltpu.VMEM_SHARED((NP, CW), jnp.float32),
        pltpu.VMEM((K, CW), jnp.float32),
        pltpu.SemaphoreType.DMA,
    ]

  @functools.partial(
      pl.kernel,
      out_type=tuple(out_type),
      mesh=mesh,
      scratch_types=scratch,
      compiler_params=pltpu.CompilerParams(use_tc_tiling_on_sc=False),
  )
  def agg(x, idx0, idx1, zeros, *rest):
    if counts:
      (zeros_c, ones, out, out_c, acc, idxr, rows, isem, gsem, ssem,
       cacc, ones_v, csem) = rest
    else:
      out, acc, idxr, rows, isem, gsem, ssem = rest
    c = lax.axis_index("c")
    s = lax.axis_index("s")
    r0 = s * RPT

    def fire_idx(q, j):
      @pl.when(c == 0)
      def _():
        pltpu.async_copy(idx0.at[0, s, j], idxr[q].at[0], isem[q])
        pltpu.async_copy(idx0.at[1, s, j], idxr[q].at[1], isem[q])

      @pl.when(c == 1)
      def _():
        pltpu.async_copy(idx1.at[0, s, j], idxr[q].at[0], isem[q])
        pltpu.async_copy(idx1.at[1, s, j], idxr[q].at[1], isem[q])

    def wait_idx(q, j):
      pltpu.make_async_copy(idx0.at[0, s, j], idxr[q].at[0], isem[q]).wait()
      pltpu.make_async_copy(idx0.at[1, s, j], idxr[q].at[1], isem[q]).wait()

    pltpu.sync_copy(zeros, acc.at[pl.ds(r0, RPT)])
    if counts:
      pltpu.sync_copy(zeros_c, cacc.at[pl.ds(r0, RPT)])
      pltpu.sync_copy(ones, ones_v)
    for q in range(NI):
      fire_idx(q, q)
    for b in range(NBUF):
      wait_idx(b, b)
      pltpu.async_copy(x.at[idxr[b].at[0]], rows[b], gsem[b])
    plsc.subcore_barrier()

    def body(i, carry):
      for u in range(NI):
        j = i * NI + u
        b = u % NBUF
        pltpu.make_async_copy(x.at[idxr[u].at[0]], rows[b], gsem[b]).wait()
        if counts:
          cd = pltpu.async_copy(ones_v, cacc.at[idxr[u].at[1]], csem,
                                add=True)
        sd = pltpu.async_copy(rows[b], acc.at[idxr[u].at[1]], ssem[b],
                              add=True)
        sd.wait()
        if counts:
          cd.wait()

        @pl.when(j + NI < C)
        def _():
          fire_idx(u, j + NI)

        q2 = (u + NBUF) % NI

        @pl.when(j + NBUF < C)
        def _():
          wait_idx(q2, j + NBUF)
          pltpu.async_copy(x.at[idxr[q2].at[0]], rows[b], gsem[b])

      return carry

    lax.fori_loop(0, C // NI, body, 0)
    plsc.subcore_barrier()
    pltpu.sync_copy(acc.at[pl.ds(r0, RPT)], out.at[c, pl.ds(r0, RPT)])
    if counts:
      pltpu.sync_copy(cacc.at[pl.ds(r0, RPT)], out_c.at[c, pl.ds(r0, RPT)])

  return agg


def _dot(a, b):
  return jnp.dot(a, b, preferred_element_type=jnp.float32,
                 precision=lax.Precision.DEFAULT)


def _tc1_body(x_ref, agg_ref, cnt_ref, wr_ref, wroot_ref, b_ref, h_ref):
  m0 = agg_ref[0] / jnp.maximum(cnt_ref[0][:, 0:1], 1.0)
  m1 = agg_ref[1] / jnp.maximum(cnt_ref[1][:, 0:1], 1.0)
  acc = _dot(x_ref[...], wroot_ref[...]) + b_ref[...]
  acc = acc + _dot(m0, wr_ref[0]) + _dot(m1, wr_ref[1])
  h_ref[...] = jnp.maximum(acc, 0.0)


def _tc2_body(h_ref, agg_ref, cnt_ref, wr_ref, wroot_ref, b_ref, out_ref):
  m0 = agg_ref[0] / jnp.maximum(cnt_ref[0][:, 0:1], 1.0)
  m1 = agg_ref[1] / jnp.maximum(cnt_ref[1][:, 0:1], 1.0)
  acc = _dot(h_ref[...], wroot_ref[...]) + b_ref[...]
  out_ref[...] = acc + _dot(m0, wr_ref[0]) + _dot(m1, wr_ref[1])


def _tc_layer(body, relu_unused):
  return pl.pallas_call(
      body,
      grid=(N // BM,),
      in_specs=[
          pl.BlockSpec((BM, D), lambda i: (i, 0)),
          pl.BlockSpec((NC, BM, D), lambda i: (0, i, 0)),
          pl.BlockSpec((NC, BM, CW), lambda i: (0, i, 0)),
          pl.BlockSpec((NC, D, D), lambda i: (0, 0, 0)),
          pl.BlockSpec((D, D), lambda i: (0, 0)),
          pl.BlockSpec((1, D), lambda i: (0, 0)),
      ],
      out_specs=pl.BlockSpec((BM, D), lambda i: (i, 0)),
      out_shape=jax.ShapeDtypeStruct((N, D), jnp.float32),
  )


def kernel(x, edge_index_r0, edge_index_r1, Wr0, Wroot0, b0, Wr1, Wroot1, b1):
  x = jnp.asarray(x, jnp.float32)
  idx0 = edge_index_r0.astype(jnp.int32).reshape(2, NS, C, K)
  idx1 = edge_index_r1.astype(jnp.int32).reshape(2, NS, C, K)

  zeros = jnp.zeros((RPT, D), jnp.float32)
  agg1, cnt = _sc_aggregate(True)(x, idx0, idx1, zeros,
                                  jnp.zeros((RPT, CW), jnp.float32),
                                  jnp.ones((K, CW), jnp.float32))
  h1 = _tc_layer(_tc1_body, True)(x, agg1, cnt, Wr0, Wroot0,
                                  b0.reshape(1, D))
  (agg2,) = _sc_aggregate(False)(h1, idx0, idx1, zeros)
  return _tc_layer(_tc2_body, False)(h1, agg2, cnt, Wr1, Wroot1,
                                     b1.reshape(1, D))


# speedup vs baseline: 1.0517x; 1.0004x over previous
import functools

import jax
import jax.numpy as jnp
from jax import lax
from jax.experimental import pallas as pl
from jax.experimental.pallas import tpu as pltpu
from jax.experimental.pallas import tpu_sc as plsc

N = 10000
D = 128
E = 160000
NC = 2
NS = 16
NP = 10240
RPT = NP // NS
EPT = E // NS
K = 50
C = EPT // K
NBUF = 4
NI = 8
CW = 16
BM = 5000


@functools.cache
def _sc_aggregate(counts):
  mesh = plsc.VectorSubcoreMesh(core_axis_name="c", subcore_axis_name="s")

  out_type = [jax.ShapeDtypeStruct((NC, NP, D), jnp.float32)]
  scratch = [
      pltpu.VMEM_SHARED((NP, D), jnp.float32),
      [pltpu.VMEM((2, K), jnp.int32) for _ in range(NI)],
      [pltpu.VMEM((K, D), jnp.float32) for _ in range(NBUF)],
      [pltpu.SemaphoreType.DMA for _ in range(NI)],
      [pltpu.SemaphoreType.DMA for _ in range(NBUF)],
      [pltpu.SemaphoreType.DMA for _ in range(NBUF)],
  ]
  if counts:
    out_type.append(jax.ShapeDtypeStruct((NC, NP, CW), jnp.float32))
    scratch += [
        pltpu.VMEM_SHARED((NP, CW), jnp.float32),
        pltpu.VMEM((K, CW), jnp.float32),
        pltpu.SemaphoreType.DMA,
    ]

  @functools.partial(
      pl.kernel,
      out_type=tuple(out_type),
      mesh=mesh,
      scratch_types=scratch,
      compiler_params=pltpu.CompilerParams(use_tc_tiling_on_sc=False),
  )
  def agg(x, idx0, idx1, zeros, *rest):
    if counts:
      (zeros_c, ones, out, out_c, acc, idxr, rows, isem, gsem, ssem,
       cacc, ones_v, csem) = rest
    else:
      out, acc, idxr, rows, isem, gsem, ssem = rest
    c = lax.axis_index("c")
    s = lax.axis_index("s")
    r0 = s * RPT

    def fire_idx(q, j):
      @pl.when(c == 0)
      def _():
        pltpu.async_copy(idx0.at[0, s, j], idxr[q].at[0], isem[q])
        pltpu.async_copy(idx0.at[1, s, j], idxr[q].at[1], isem[q])

      @pl.when(c == 1)
      def _():
        pltpu.async_copy(idx1.at[0, s, j], idxr[q].at[0], isem[q])
        pltpu.async_copy(idx1.at[1, s, j], idxr[q].at[1], isem[q])

    def wait_idx(q, j):
      pltpu.make_async_copy(idx0.at[0, s, j], idxr[q].at[0], isem[q]).wait()
      pltpu.make_async_copy(idx0.at[1, s, j], idxr[q].at[1], isem[q]).wait()

    pltpu.sync_copy(zeros, acc.at[pl.ds(r0, RPT)])
    if counts:
      pltpu.sync_copy(zeros_c, cacc.at[pl.ds(r0, RPT)])
      pltpu.sync_copy(ones, ones_v)
    for q in range(NI):
      fire_idx(q, q)
    for b in range(NBUF):
      wait_idx(b, b)
      pltpu.async_copy(x.at[idxr[b].at[0]], rows[b], gsem[b])
    plsc.subcore_barrier()

    def body(i, carry):
      for u in range(NI):
        j = i * NI + u
        b = u % NBUF
        pltpu.make_async_copy(x.at[idxr[u].at[0]], rows[b], gsem[b]).wait()
        if counts:
          cd = pltpu.async_copy(ones_v, cacc.at[idxr[u].at[1]], csem,
                                add=True)
        sd = pltpu.async_copy(rows[b], acc.at[idxr[u].at[1]], ssem[b],
                              add=True)
        sd.wait()
        if counts:
          cd.wait()

        @pl.when(j + NI < C)
        def _():
          fire_idx(u, j + NI)

        q2 = (u + NBUF) % NI

        @pl.when(j + NBUF < C)
        def _():
          wait_idx(q2, j + NBUF)
          pltpu.async_copy(x.at[idxr[q2].at[0]], rows[b], gsem[b])

      return carry

    lax.fori_loop(0, C // NI, body, 0)
    plsc.subcore_barrier()
    pltpu.sync_copy(acc.at[pl.ds(r0, RPT)], out.at[c, pl.ds(r0, RPT)])
    if counts:
      pltpu.sync_copy(cacc.at[pl.ds(r0, RPT)], out_c.at[c, pl.ds(r0, RPT)])

  return agg


def _dot(a, b):
  return jnp.dot(a, b, preferred_element_type=jnp.float32,
                 precision=lax.Precision.DEFAULT)


def _tc1_body(x_ref, agg_ref, cnt_ref, wr_ref, wroot_ref, b_ref, h_ref):
  m0 = agg_ref[0] / jnp.maximum(cnt_ref[0][:, 0:1], 1.0)
  m1 = agg_ref[1] / jnp.maximum(cnt_ref[1][:, 0:1], 1.0)
  acc = _dot(x_ref[...], wroot_ref[...]) + b_ref[...]
  acc = acc + _dot(m0, wr_ref[0]) + _dot(m1, wr_ref[1])
  h_ref[...] = jnp.maximum(acc, 0.0)


def _tc2_body(h_ref, agg_ref, cnt_ref, wr_ref, wroot_ref, b_ref, out_ref):
  m0 = agg_ref[0] / jnp.maximum(cnt_ref[0][:, 0:1], 1.0)
  m1 = agg_ref[1] / jnp.maximum(cnt_ref[1][:, 0:1], 1.0)
  acc = _dot(h_ref[...], wroot_ref[...]) + b_ref[...]
  out_ref[...] = acc + _dot(m0, wr_ref[0]) + _dot(m1, wr_ref[1])


def _tc_layer(body, relu_unused):
  return pl.pallas_call(
      body,
      grid=(N // BM,),
      in_specs=[
          pl.BlockSpec((BM, D), lambda i: (i, 0)),
          pl.BlockSpec((NC, BM, D), lambda i: (0, i, 0)),
          pl.BlockSpec((NC, BM, CW), lambda i: (0, i, 0)),
          pl.BlockSpec((NC, D, D), lambda i: (0, 0, 0)),
          pl.BlockSpec((D, D), lambda i: (0, 0)),
          pl.BlockSpec((1, D), lambda i: (0, 0)),
      ],
      out_specs=pl.BlockSpec((BM, D), lambda i: (i, 0)),
      out_shape=jax.ShapeDtypeStruct((N, D), jnp.float32),
  )


def kernel(x, edge_index_r0, edge_index_r1, Wr0, Wroot0, b0, Wr1, Wroot1, b1):
  x = jnp.asarray(x, jnp.float32)
  idx0 = edge_index_r0.astype(jnp.int32).reshape(2, NS, C, K)
  idx1 = edge_index_r1.astype(jnp.int32).reshape(2, NS, C, K)

  zeros = jnp.zeros((RPT, D), jnp.float32)
  agg1, cnt = _sc_aggregate(True)(x, idx0, idx1, zeros,
                                  jnp.zeros((RPT, CW), jnp.float32),
                                  jnp.ones((K, CW), jnp.float32))
  h1 = _tc_layer(_tc1_body, True)(x, agg1, cnt, Wr0, Wroot0,
                                  b0.reshape(1, D))
  (agg2,) = _sc_aggregate(False)(h1, idx0, idx1, zeros)
  return _tc_layer(_tc2_body, False)(h1, agg2, cnt, Wr1, Wroot1,
                                     b1.reshape(1, D))
